# Initial kernel scaffold; baseline (speedup 1.0000x reference)
import jax
import jax.numpy as jnp
from jax.experimental import pallas as pl


def kernel(x, gate_w, expert_gate_w, expert_up_w, expert_down_w, shared_gate_w, shared_up_w, shared_down_w):
    raise NotImplementedError("write your pallas kernel here")


# speedup vs baseline: 1.2610x; 1.2610x over previous
import functools

import jax
import jax.numpy as jnp
from jax.experimental import pallas as pl
from jax.experimental.pallas import tpu as pltpu

DIM = 1024
INTER = 512
NE = 16
TOPK = 2
NG = 18
M = 256


def _gmm_body(ms_ref, mx_ref, mo_ref, ew_ref, ec_ref, sh_ref, fi_ref,
              a_ref, x_ref, g_ref, w_ref, gw_ref, up_ref, dw_ref, out_ref):
    t = pl.program_id(0)
    e = ec_ref[t]
    a = jnp.where(sh_ref[t] == 1, x_ref[...], a_ref[...])
    g = g_ref[0, 0, :]
    w = w_ref[0, 0, :]
    wm = jnp.where(g == e, w, 0.0)
    hg = jax.lax.dot_general(a, gw_ref[0], (((1,), (1,)), ((), ())),
                             preferred_element_type=jnp.float32)
    hu = jax.lax.dot_general(a, up_ref[0], (((1,), (1,)), ((), ())),
                             preferred_element_type=jnp.float32)
    h = hg * jax.lax.logistic(hg) * hu * wm[:, None]
    contrib = jax.lax.dot_general(h, dw_ref[0], (((1,), (1,)), ((), ())),
                                  preferred_element_type=jnp.float32)

    @pl.when(fi_ref[t] == 1)
    def _():
        out_ref[...] = contrib

    @pl.when(fi_ref[t] == 0)
    def _():
        out_ref[...] += contrib


def kernel(x, gate_w, expert_gate_w, expert_up_w, expert_down_w,
           shared_gate_w, shared_up_w, shared_down_w):
    shape = x.shape
    xf = x.reshape(-1, DIM).astype(jnp.float32)
    nt = xf.shape[0]
    nr = nt * TOPK
    rtot = nr + 2 * nt
    nb_r = nr // M
    nb_x = nt // M
    nb = rtot // M
    steps = nb_r + (NE - 1) + 2 * nb_x

    scores = jax.nn.softmax(xf @ gate_w.astype(jnp.float32).T, axis=-1)
    topw, topi = jax.lax.top_k(scores, TOPK)

    flat_e = topi.reshape(-1).astype(jnp.int32)
    flat_w = topw.reshape(-1)
    onehot = (flat_e[:, None] == jnp.arange(NE, dtype=jnp.int32)[None, :])
    counts = jnp.sum(onehot.astype(jnp.int32), axis=0)
    within = jnp.cumsum(onehot.astype(jnp.int32), axis=0) - onehot.astype(jnp.int32)
    rank = jnp.take_along_axis(within, flat_e[:, None], axis=1)[:, 0]
    off = jnp.concatenate([jnp.zeros((1,), jnp.int32), jnp.cumsum(counts)[:-1]])
    pos = off[flat_e] + rank
    tok = (jnp.arange(nr, dtype=jnp.int32) // TOPK)
    z = jnp.zeros((nr,), jnp.int32)
    sort_tok = z.at[pos].set(tok)
    sort_g = z.at[pos].set(flat_e)
    sort_w = jnp.zeros((nr,), jnp.float32).at[pos].set(flat_w)

    a_sorted = jnp.take(xf, sort_tok, axis=0)

    g_all = jnp.concatenate([
        sort_g,
        jnp.full((nt,), NE, jnp.int32),
        jnp.full((nt,), NE + 1, jnp.int32),
    ]).reshape(nb, 1, M)
    w_all = jnp.concatenate([sort_w, jnp.ones((2 * nt,), jnp.float32)]
                            ).reshape(nb, 1, M)

    sizes = jnp.concatenate([counts, jnp.array([nt, nt], jnp.int32)])
    off18 = jnp.concatenate([jnp.zeros((1,), jnp.int32),
                             jnp.cumsum(sizes)[:-1]]).astype(jnp.int32)
    ends = off18 + sizes
    first_blk = off18 // M
    last_blk = (ends - 1) // M
    tiles = jnp.where(sizes > 0, last_blk - first_blk + 1, 0)
    ctiles = jnp.cumsum(tiles)
    step_start = ctiles - tiles
    t_ar = jnp.arange(steps, dtype=jnp.int32)
    e_arr = jnp.searchsorted(ctiles, t_ar, side='right').astype(jnp.int32)
    e_cl = jnp.minimum(e_arr, NG - 1)
    j = t_ar - step_start[e_cl]
    valid = e_arr < NG
    m_glob = jnp.where(valid, first_blk[e_cl] + j, nb - 1)
    ms = jnp.minimum(m_glob, nb_r - 1)
    mx = jnp.where(e_arr == NE, m_glob - nb_r,
                   jnp.where(e_arr == NE + 1, m_glob - nb_r - nb_x,
                             jnp.where(e_arr > NE + 1, nb_x - 1, 0)))
    mo = m_glob
    ew = e_cl
    ec = jnp.minimum(e_arr, NG)
    sh = (e_arr >= NE).astype(jnp.int32)
    prev_mo = jnp.concatenate([jnp.array([-1], jnp.int32), mo[:-1]])
    fi = jnp.logical_and(mo != prev_mo, valid).astype(jnp.int32)

    gw_all = jnp.concatenate(
        [expert_gate_w, shared_gate_w.reshape(2, INTER, DIM)], axis=0
    ).astype(jnp.float32)
    up_all = jnp.concatenate(
        [expert_up_w, shared_up_w.reshape(2, INTER, DIM)], axis=0
    ).astype(jnp.float32)
    dw_all = jnp.concatenate(
        [expert_down_w,
         jnp.stack([shared_down_w[:, :INTER], shared_down_w[:, INTER:]])],
        axis=0).astype(jnp.float32)

    grid_spec = pltpu.PrefetchScalarGridSpec(
        num_scalar_prefetch=7,
        grid=(steps,),
        in_specs=[
            pl.BlockSpec((M, DIM), lambda t, ms, mx, mo, ew, ec, sh, fi: (ms[t], 0)),
            pl.BlockSpec((M, DIM), lambda t, ms, mx, mo, ew, ec, sh, fi: (mx[t], 0)),
            pl.BlockSpec((1, 1, M), lambda t, ms, mx, mo, ew, ec, sh, fi: (mo[t], 0, 0)),
            pl.BlockSpec((1, 1, M), lambda t, ms, mx, mo, ew, ec, sh, fi: (mo[t], 0, 0)),
            pl.BlockSpec((1, INTER, DIM), lambda t, ms, mx, mo, ew, ec, sh, fi: (ew[t], 0, 0)),
            pl.BlockSpec((1, INTER, DIM), lambda t, ms, mx, mo, ew, ec, sh, fi: (ew[t], 0, 0)),
            pl.BlockSpec((1, DIM, INTER), lambda t, ms, mx, mo, ew, ec, sh, fi: (ew[t], 0, 0)),
        ],
        out_specs=pl.BlockSpec((M, DIM), lambda t, ms, mx, mo, ew, ec, sh, fi: (mo[t], 0)),
    )
    out = pl.pallas_call(
        _gmm_body,
        grid_spec=grid_spec,
        out_shape=jax.ShapeDtypeStruct((rtot, DIM), jnp.float32),
        compiler_params=pltpu.CompilerParams(
            dimension_semantics=("arbitrary",)),
    )(ms, mx, mo, ew, ec, sh, fi,
      a_sorted, xf, g_all, w_all, gw_all, up_all, dw_all)

    p = pos.reshape(nt, TOPK)
    y = (jnp.take(out, p[:, 0], axis=0) + jnp.take(out, p[:, 1], axis=0)
         + out[nr:nr + nt] + out[nr + nt:])
    return y.astype(x.dtype).reshape(shape)


# speedup vs baseline: 1.2893x; 1.0225x over previous
import functools

import jax
import jax.numpy as jnp
from jax.experimental import pallas as pl
from jax.experimental.pallas import tpu as pltpu

DIM = 1024
INTER = 512
NE = 16
TOPK = 2
NG = 18
M = 256


def _gmm_body(ms_ref, mx_ref, mo_ref, ew_ref, ec_ref, sh_ref, fi_ref,
              a_ref, x_ref, g_ref, w_ref, gw_ref, up_ref, dw_ref, out_ref):
    t = pl.program_id(0)
    e = ec_ref[t]
    a = jnp.where(sh_ref[t] == 1, x_ref[...], a_ref[...])
    g = g_ref[0, 0, :]
    w = w_ref[0, 0, :]
    wm = jnp.where(g == e, w, 0.0)
    hg = jax.lax.dot_general(a, gw_ref[0], (((1,), (1,)), ((), ())),
                             preferred_element_type=jnp.float32)
    hu = jax.lax.dot_general(a, up_ref[0], (((1,), (1,)), ((), ())),
                             preferred_element_type=jnp.float32)
    h = (hg * jax.lax.logistic(hg) * hu * wm[:, None]).astype(jnp.bfloat16)
    contrib = jax.lax.dot_general(h, dw_ref[0], (((1,), (1,)), ((), ())),
                                  preferred_element_type=jnp.float32)

    @pl.when(fi_ref[t] == 1)
    def _():
        out_ref[...] = contrib

    @pl.when(fi_ref[t] == 0)
    def _():
        out_ref[...] += contrib


def kernel(x, gate_w, expert_gate_w, expert_up_w, expert_down_w,
           shared_gate_w, shared_up_w, shared_down_w):
    shape = x.shape
    xf = x.reshape(-1, DIM).astype(jnp.float32)
    nt = xf.shape[0]
    nr = nt * TOPK
    rtot = nr + 2 * nt
    nb_r = nr // M
    nb_x = nt // M
    nb = rtot // M
    steps = nb_r + (NE - 1) + 2 * nb_x

    scores = jax.nn.softmax(xf @ gate_w.astype(jnp.float32).T, axis=-1)
    topw, topi = jax.lax.top_k(scores, TOPK)

    flat_e = topi.reshape(-1).astype(jnp.int32)
    flat_w = topw.reshape(-1)
    onehot = (flat_e[:, None] == jnp.arange(NE, dtype=jnp.int32)[None, :])
    counts = jnp.sum(onehot.astype(jnp.int32), axis=0)
    within = jnp.cumsum(onehot.astype(jnp.int32), axis=0) - onehot.astype(jnp.int32)
    rank = jnp.take_along_axis(within, flat_e[:, None], axis=1)[:, 0]
    off = jnp.concatenate([jnp.zeros((1,), jnp.int32), jnp.cumsum(counts)[:-1]])
    pos = off[flat_e] + rank
    tok = (jnp.arange(nr, dtype=jnp.int32) // TOPK)
    z = jnp.zeros((nr,), jnp.int32)
    sort_tok = z.at[pos].set(tok)
    sort_g = z.at[pos].set(flat_e)
    sort_w = jnp.zeros((nr,), jnp.float32).at[pos].set(flat_w)

    xb = xf.astype(jnp.bfloat16)
    a_sorted = jnp.take(xb, sort_tok, axis=0)

    g_all = jnp.concatenate([
        sort_g,
        jnp.full((nt,), NE, jnp.int32),
        jnp.full((nt,), NE + 1, jnp.int32),
    ]).reshape(nb, 1, M)
    w_all = jnp.concatenate([sort_w, jnp.ones((2 * nt,), jnp.float32)]
                            ).reshape(nb, 1, M)

    sizes = jnp.concatenate([counts, jnp.array([nt, nt], jnp.int32)])
    off18 = jnp.concatenate([jnp.zeros((1,), jnp.int32),
                             jnp.cumsum(sizes)[:-1]]).astype(jnp.int32)
    ends = off18 + sizes
    first_blk = off18 // M
    last_blk = (ends - 1) // M
    tiles = jnp.where(sizes > 0, last_blk - first_blk + 1, 0)
    ctiles = jnp.cumsum(tiles)
    step_start = ctiles - tiles
    t_ar = jnp.arange(steps, dtype=jnp.int32)
    e_arr = jnp.searchsorted(ctiles, t_ar, side='right').astype(jnp.int32)
    e_cl = jnp.minimum(e_arr, NG - 1)
    j = t_ar - step_start[e_cl]
    valid = e_arr < NG
    m_glob = jnp.where(valid, first_blk[e_cl] + j, nb - 1)
    ms = jnp.minimum(m_glob, nb_r - 1)
    mx = jnp.where(e_arr == NE, m_glob - nb_r,
                   jnp.where(e_arr == NE + 1, m_glob - nb_r - nb_x,
                             jnp.where(e_arr > NE + 1, nb_x - 1, 0)))
    mo = m_glob
    ew = e_cl
    ec = jnp.minimum(e_arr, NG)
    sh = (e_arr >= NE).astype(jnp.int32)
    prev_mo = jnp.concatenate([jnp.array([-1], jnp.int32), mo[:-1]])
    fi = jnp.logical_and(mo != prev_mo, valid).astype(jnp.int32)

    gw_all = jnp.concatenate(
        [expert_gate_w, shared_gate_w.reshape(2, INTER, DIM)], axis=0
    ).astype(jnp.bfloat16)
    up_all = jnp.concatenate(
        [expert_up_w, shared_up_w.reshape(2, INTER, DIM)], axis=0
    ).astype(jnp.bfloat16)
    dw_all = jnp.concatenate(
        [expert_down_w,
         jnp.stack([shared_down_w[:, :INTER], shared_down_w[:, INTER:]])],
        axis=0).astype(jnp.bfloat16)

    grid_spec = pltpu.PrefetchScalarGridSpec(
        num_scalar_prefetch=7,
        grid=(steps,),
        in_specs=[
            pl.BlockSpec((M, DIM), lambda t, ms, mx, mo, ew, ec, sh, fi: (ms[t], 0)),
            pl.BlockSpec((M, DIM), lambda t, ms, mx, mo, ew, ec, sh, fi: (mx[t], 0)),
            pl.BlockSpec((1, 1, M), lambda t, ms, mx, mo, ew, ec, sh, fi: (mo[t], 0, 0)),
            pl.BlockSpec((1, 1, M), lambda t, ms, mx, mo, ew, ec, sh, fi: (mo[t], 0, 0)),
            pl.BlockSpec((1, INTER, DIM), lambda t, ms, mx, mo, ew, ec, sh, fi: (ew[t], 0, 0)),
            pl.BlockSpec((1, INTER, DIM), lambda t, ms, mx, mo, ew, ec, sh, fi: (ew[t], 0, 0)),
            pl.BlockSpec((1, DIM, INTER), lambda t, ms, mx, mo, ew, ec, sh, fi: (ew[t], 0, 0)),
        ],
        out_specs=pl.BlockSpec((M, DIM), lambda t, ms, mx, mo, ew, ec, sh, fi: (mo[t], 0)),
    )
    out = pl.pallas_call(
        _gmm_body,
        grid_spec=grid_spec,
        out_shape=jax.ShapeDtypeStruct((rtot, DIM), jnp.float32),
        compiler_params=pltpu.CompilerParams(
            dimension_semantics=("arbitrary",)),
    )(ms, mx, mo, ew, ec, sh, fi,
      a_sorted, xb, g_all, w_all, gw_all, up_all, dw_all)

    p = pos.reshape(nt, TOPK)
    y = (jnp.take(out, p[:, 0], axis=0) + jnp.take(out, p[:, 1], axis=0)
         + out[nr:nr + nt] + out[nr + nt:])
    return y.astype(x.dtype).reshape(shape)


# speedup vs baseline: 1.5015x; 1.1646x over previous
import jax
import jax.numpy as jnp
from jax.experimental import pallas as pl
from jax.experimental.pallas import tpu as pltpu

DIM = 1024
INTER = 512
NE = 16
TOPK = 2
NG = 18
M = 256
CH = 128


def _router_body(x_ref, gw_ref, pos_ref, wts_ref, cnt_ref, cum_ref, oh_ref):
    s = jax.lax.dot_general(x_ref[...], gw_ref[...], (((1,), (1,)), ((), ())),
                            preferred_element_type=jnp.float32)
    nt = s.shape[0]
    m = jnp.max(s, axis=1, keepdims=True)
    p = jnp.exp(s - m)
    sm = p / jnp.sum(p, axis=1, keepdims=True)
    lane = jax.lax.broadcasted_iota(jnp.int32, (nt, NE), 1)
    m1 = jnp.max(sm, axis=1, keepdims=True)
    i1 = jnp.min(jnp.where(sm == m1, lane, NE), axis=1, keepdims=True)
    sm2 = jnp.where(lane == i1, -1.0, sm)
    m2 = jnp.max(sm2, axis=1, keepdims=True)
    i2 = jnp.min(jnp.where(sm2 == m2, lane, NE), axis=1, keepdims=True)
    oh_ref[...] = ((lane == i1) | (lane == i2)).astype(jnp.float32)

    r = jax.lax.broadcasted_iota(jnp.int32, (CH, CH), 0)
    c = jax.lax.broadcasted_iota(jnp.int32, (CH, CH), 1)
    tri = (r >= c).astype(jnp.float32)

    def step(i, carry):
        ch = oh_ref[pl.ds(i * CH, CH), :]
        incl = jax.lax.dot_general(tri, ch, (((1,), (0,)), ((), ())),
                                   preferred_element_type=jnp.float32)
        cum_ref[pl.ds(i * CH, CH), :] = incl - ch + carry
        return carry + incl[CH - 1:CH, :]

    counts = jax.lax.fori_loop(0, nt // CH, step, jnp.zeros((1, NE), jnp.float32))

    lane1 = lane[0:1, :]
    off = jnp.zeros((1, NE), jnp.float32)
    for k in range(NE):
        ck = jnp.sum(jnp.where(lane1 == k, counts, 0.0), axis=1, keepdims=True)
        off = off + jnp.where(lane1 > k, ck, 0.0)

    cum = cum_ref[...]
    offb = jnp.broadcast_to(off, (nt, NE))
    pos1 = jnp.sum(jnp.where(lane == i1, cum + offb, 0.0), axis=1, keepdims=True)
    pos2 = jnp.sum(jnp.where(lane == i2, cum + offb, 0.0), axis=1, keepdims=True)
    pos_ref[...] = jnp.concatenate([pos1, pos2], axis=1).astype(jnp.int32)
    wts_ref[...] = jnp.concatenate([m1, m2], axis=1)
    cnt_ref[...] = jnp.broadcast_to(counts, (8, NE)).astype(jnp.int32)


def _gmm_body(ms_ref, mx_ref, mo_ref, ew_ref, lo_ref, hi_ref, sh_ref, fi_ref,
              a_ref, x_ref, gw_ref, up_ref, dw_ref, out_ref):
    t = pl.program_id(0)
    a = jnp.where(sh_ref[t] == 1, x_ref[...], a_ref[...])
    rows = mo_ref[t] * M + jax.lax.broadcasted_iota(jnp.int32, (M, 1), 0)
    mask = ((rows >= lo_ref[t]) & (rows < hi_ref[t])).astype(jnp.float32)
    hg = jax.lax.dot_general(a, gw_ref[0], (((1,), (1,)), ((), ())),
                             preferred_element_type=jnp.float32)
    hu = jax.lax.dot_general(a, up_ref[0], (((1,), (1,)), ((), ())),
                             preferred_element_type=jnp.float32)
    h = (hg * jax.lax.logistic(hg) * hu * mask).astype(jnp.bfloat16)
    contrib = jax.lax.dot_general(h, dw_ref[0], (((1,), (1,)), ((), ())),
                                  preferred_element_type=jnp.float32)

    @pl.when(fi_ref[t] == 1)
    def _():
        out_ref[...] = contrib

    @pl.when(fi_ref[t] == 0)
    def _():
        out_ref[...] += contrib


def kernel(x, gate_w, expert_gate_w, expert_up_w, expert_down_w,
           shared_gate_w, shared_up_w, shared_down_w):
    shape = x.shape
    xf = x.reshape(-1, DIM).astype(jnp.float32)
    nt = xf.shape[0]
    nr = nt * TOPK
    rtot = nr + 2 * nt
    nb_r = nr // M
    nb_x = nt // M
    nb = rtot // M
    steps = nb_r + (NE - 1) + 2 * nb_x

    pos, wts, cnt8 = pl.pallas_call(
        _router_body,
        out_shape=(
            jax.ShapeDtypeStruct((nt, TOPK), jnp.int32),
            jax.ShapeDtypeStruct((nt, TOPK), jnp.float32),
            jax.ShapeDtypeStruct((8, NE), jnp.int32),
        ),
        scratch_shapes=[pltpu.VMEM((nt, NE), jnp.float32),
                        pltpu.VMEM((nt, NE), jnp.float32)],
    )(xf, gate_w.astype(jnp.float32))
    counts = cnt8[0]

    flat_pos = pos.reshape(-1)
    sorted_a = jnp.zeros((nr,), jnp.int32).at[flat_pos].set(
        jnp.arange(nr, dtype=jnp.int32))
    xb = xf.astype(jnp.bfloat16)
    a_sorted = jnp.take(xb, sorted_a >> 1, axis=0)

    sizes = jnp.concatenate([counts, jnp.array([nt, nt], jnp.int32)])
    off18 = jnp.concatenate([jnp.zeros((1,), jnp.int32),
                             jnp.cumsum(sizes)[:-1]]).astype(jnp.int32)
    ends = off18 + sizes
    first_blk = off18 // M
    last_blk = (ends - 1) // M
    tiles = jnp.where(sizes > 0, last_blk - first_blk + 1, 0)
    ctiles = jnp.cumsum(tiles)
    step_start = ctiles - tiles
    t_ar = jnp.arange(steps, dtype=jnp.int32)
    e_arr = jnp.searchsorted(ctiles, t_ar, side='right').astype(jnp.int32)
    e_cl = jnp.minimum(e_arr, NG - 1)
    j = t_ar - step_start[e_cl]
    valid = e_arr < NG
    m_glob = jnp.where(valid, first_blk[e_cl] + j, nb - 1)
    ms = jnp.minimum(m_glob, nb_r - 1)
    mx = jnp.where(e_arr == NE, m_glob - nb_r,
                   jnp.where(e_arr == NE + 1, m_glob - nb_r - nb_x,
                             jnp.where(e_arr > NE + 1, nb_x - 1, 0)))
    mo = m_glob
    ew = e_cl
    lo = jnp.where(valid, off18[e_cl], 0)
    hi = jnp.where(valid, ends[e_cl], 0)
    sh = (e_arr >= NE).astype(jnp.int32)
    prev_mo = jnp.concatenate([jnp.array([-1], jnp.int32), mo[:-1]])
    fi = jnp.logical_and(mo != prev_mo, valid).astype(jnp.int32)

    gw_all = jnp.concatenate(
        [expert_gate_w, shared_gate_w.reshape(2, INTER, DIM)], axis=0
    ).astype(jnp.bfloat16)
    up_all = jnp.concatenate(
        [expert_up_w, shared_up_w.reshape(2, INTER, DIM)], axis=0
    ).astype(jnp.bfloat16)
    dw_all = jnp.concatenate(
        [expert_down_w,
         jnp.stack([shared_down_w[:, :INTER], shared_down_w[:, INTER:]])],
        axis=0).astype(jnp.bfloat16)

    grid_spec = pltpu.PrefetchScalarGridSpec(
        num_scalar_prefetch=8,
        grid=(steps,),
        in_specs=[
            pl.BlockSpec((M, DIM), lambda t, ms, mx, mo, ew, lo, hi, sh, fi: (ms[t], 0)),
            pl.BlockSpec((M, DIM), lambda t, ms, mx, mo, ew, lo, hi, sh, fi: (mx[t], 0)),
            pl.BlockSpec((1, INTER, DIM), lambda t, ms, mx, mo, ew, lo, hi, sh, fi: (ew[t], 0, 0)),
            pl.BlockSpec((1, INTER, DIM), lambda t, ms, mx, mo, ew, lo, hi, sh, fi: (ew[t], 0, 0)),
            pl.BlockSpec((1, DIM, INTER), lambda t, ms, mx, mo, ew, lo, hi, sh, fi: (ew[t], 0, 0)),
        ],
        out_specs=pl.BlockSpec((M, DIM), lambda t, ms, mx, mo, ew, lo, hi, sh, fi: (mo[t], 0)),
    )
    out = pl.pallas_call(
        _gmm_body,
        grid_spec=grid_spec,
        out_shape=jax.ShapeDtypeStruct((rtot, DIM), jnp.float32),
        compiler_params=pltpu.CompilerParams(
            dimension_semantics=("arbitrary",)),
    )(ms, mx, mo, ew, lo, hi, sh, fi,
      a_sorted, xb, gw_all, up_all, dw_all)

    y = (wts[:, 0:1] * jnp.take(out, pos[:, 0], axis=0)
         + wts[:, 1:2] * jnp.take(out, pos[:, 1], axis=0)
         + out[nr:nr + nt] + out[nr + nt:])
    return y.astype(x.dtype).reshape(shape)


# speedup vs baseline: 1.8182x; 1.2109x over previous
import jax
import jax.numpy as jnp
from jax.experimental import pallas as pl
from jax.experimental.pallas import tpu as pltpu

DIM = 1024
INTER = 512
NE = 16
TOPK = 2
NG = 18
M = 256
MSH = M.bit_length() - 1
CH = 128
NMETA = 8


def _router_body(x_ref, gw_ref, pos_ref, wts_ref, meta_ref, cum_ref, oh_ref):
    nt = x_ref.shape[0]
    s = jax.lax.dot_general(x_ref[...], gw_ref[...], (((1,), (1,)), ((), ())),
                            preferred_element_type=jnp.float32)
    m = jnp.max(s, axis=1, keepdims=True)
    p = jnp.exp(s - m)
    sm = p / jnp.sum(p, axis=1, keepdims=True)
    lane = jax.lax.broadcasted_iota(jnp.int32, (nt, NE), 1)
    m1 = jnp.max(sm, axis=1, keepdims=True)
    i1 = jnp.min(jnp.where(sm == m1, lane, NE), axis=1, keepdims=True)
    sm2 = jnp.where(lane == i1, -1.0, sm)
    m2 = jnp.max(sm2, axis=1, keepdims=True)
    i2 = jnp.min(jnp.where(sm2 == m2, lane, NE), axis=1, keepdims=True)
    oh_ref[...] = ((lane == i1) | (lane == i2)).astype(jnp.float32)

    r = jax.lax.broadcasted_iota(jnp.int32, (CH, CH), 0)
    c = jax.lax.broadcasted_iota(jnp.int32, (CH, CH), 1)
    tri = (r >= c).astype(jnp.float32)

    def step(i, carry):
        ch = oh_ref[pl.ds(i * CH, CH), :]
        incl = jax.lax.dot_general(tri, ch, (((1,), (0,)), ((), ())),
                                   preferred_element_type=jnp.float32)
        cum_ref[pl.ds(i * CH, CH), :] = incl - ch + carry
        return carry + incl[CH - 1:CH, :]

    counts = jax.lax.fori_loop(0, nt // CH, step, jnp.zeros((1, NE), jnp.float32))

    lane1 = lane[0:1, :]
    off = jnp.zeros((1, NE), jnp.float32)
    for k in range(NE):
        ck = jnp.sum(jnp.where(lane1 == k, counts, 0.0), axis=1, keepdims=True)
        off = off + jnp.where(lane1 > k, ck, 0.0)

    cum = cum_ref[...]
    offb = jnp.broadcast_to(off, (nt, NE))
    pos1 = jnp.sum(jnp.where(lane == i1, cum + offb, 0.0), axis=1, keepdims=True)
    pos2 = jnp.sum(jnp.where(lane == i2, cum + offb, 0.0), axis=1, keepdims=True)
    pos_ref[...] = jnp.concatenate([pos1, pos2], axis=1).astype(jnp.int32)
    wts_ref[...] = jnp.concatenate([m1, m2], axis=1)

    nr = nt * TOPK
    nb_r = nr // M
    nb_x = nt // M
    nb = (nr + 2 * nt) // M
    steps = nb_r + (NE - 1) + 2 * nb_x
    gl = jax.lax.broadcasted_iota(jnp.int32, (1, 32), 1)
    cnt_i = jnp.zeros((1, 32), jnp.int32)
    for k in range(NE):
        ck = jnp.sum(jnp.where(lane1 == k, counts, 0.0), axis=1, keepdims=True)
        cnt_i = cnt_i + jnp.where(gl == k, ck.astype(jnp.int32), 0)
    sizes = jnp.where(gl < NE, cnt_i,
                      jnp.where(gl < NG, nt, 0))
    offg = jnp.zeros((1, 32), jnp.int32)
    for k in range(NG):
        ck = jnp.sum(jnp.where(gl == k, sizes, 0), axis=1, keepdims=True)
        offg = offg + jnp.where(gl > k, ck, 0)
    endg = offg + sizes
    fblk = offg >> MSH
    lblk = (endg - 1) >> MSH
    tiles = jnp.where((sizes > 0) & (gl < NG), lblk - fblk + 1, 0)
    ctiles = jnp.zeros((1, 32), jnp.int32)
    for k in range(NG):
        ck = jnp.sum(jnp.where(gl == k, tiles, 0), axis=1, keepdims=True)
        ctiles = ctiles + jnp.where(gl >= k, ck, 0)
    sstart = ctiles - tiles

    tcol = jax.lax.broadcasted_iota(jnp.int32, (CH, 1), 0)
    big = jnp.broadcast_to(jnp.where(gl < NG, ctiles, 10 ** 9), (CH, 32))
    e_arr = jnp.sum((big <= tcol).astype(jnp.int32), axis=1, keepdims=True)

    def glut(v):
        return jnp.sum(jnp.where(
            jax.lax.broadcasted_iota(jnp.int32, (CH, 32), 1) == e_arr,
            jnp.broadcast_to(v, (CH, 32)), 0), axis=1, keepdims=True)

    valid = (e_arr < NG) & (tcol < steps)
    j = tcol - glut(sstart)
    m_glob = jnp.where(valid, glut(fblk) + j, nb - 1)
    ms = jnp.minimum(m_glob, nb_r - 1)
    mx = jnp.where(valid & (e_arr == NE), m_glob - nb_r,
                   jnp.where(valid & (e_arr == NE + 1), m_glob - nb_r - nb_x,
                             jnp.where(e_arr > NE, nb_x - 1, 0)))
    ew = jnp.minimum(e_arr, NE - 1)
    ews = jnp.where(e_arr >= NE, jnp.minimum(e_arr - NE, 1), 0)
    lo = jnp.where(valid, glut(offg), 0)
    hi = jnp.where(valid, glut(endg), 0)
    sh = (e_arr >= NE).astype(jnp.int32)
    fi = (valid & ((j > 0) | ((glut(offg) & (M - 1)) == 0))).astype(jnp.int32)

    mcol = jax.lax.broadcasted_iota(jnp.int32, (CH, NMETA), 1)
    meta = jnp.where(mcol == 0, ms, 0)
    for k, v in enumerate([mx, m_glob, ew, lo, hi, ews, fi]):
        meta = meta + jnp.where(mcol == k + 1, v, 0)
    meta_ref[...] = jnp.where(mcol == 7, jnp.where(sh == 1, fi + 2, fi), meta)


def _gmm_body(meta_ref, a_ref, x_ref, gw_ref, up_ref, dw_ref,
              gws_ref, ups_ref, dws_ref, out_ref):
    t = pl.program_id(0)
    rows = meta_ref[t, 2] * M + jax.lax.broadcasted_iota(jnp.int32, (M, 1), 0)
    mask = ((rows >= meta_ref[t, 4]) & (rows < meta_ref[t, 5])
            ).astype(jnp.float32)
    code = meta_ref[t, 7]

    def ffn(a, gw, up, dw):
        hg = jax.lax.dot_general(a, gw, (((1,), (1,)), ((), ())),
                                 preferred_element_type=jnp.float32)
        hu = jax.lax.dot_general(a, up, (((1,), (1,)), ((), ())),
                                 preferred_element_type=jnp.float32)
        h = hg * jax.lax.logistic(hg) * hu * mask
        return jax.lax.dot_general(h, dw, (((1,), (1,)), ((), ())),
                                   preferred_element_type=jnp.float32)

    @pl.when(code == 0)
    def _():
        out_ref[...] += ffn(a_ref[...], gw_ref[0], up_ref[0], dw_ref[0])

    @pl.when(code == 1)
    def _():
        out_ref[...] = ffn(a_ref[...], gw_ref[0], up_ref[0], dw_ref[0])

    @pl.when(code == 2)
    def _():
        out_ref[...] += ffn(x_ref[...], gws_ref[0], ups_ref[0], dws_ref[0])

    @pl.when(code == 3)
    def _():
        out_ref[...] = ffn(x_ref[...], gws_ref[0], ups_ref[0], dws_ref[0])


def kernel(x, gate_w, expert_gate_w, expert_up_w, expert_down_w,
           shared_gate_w, shared_up_w, shared_down_w):
    shape = x.shape
    xf = x.reshape(-1, DIM).astype(jnp.float32)
    nt = xf.shape[0]
    nr = nt * TOPK
    rtot = nr + 2 * nt
    nb_r = nr // M
    nb_x = nt // M
    steps = nb_r + (NE - 1) + 2 * nb_x

    pos, wts, meta = pl.pallas_call(
        _router_body,
        out_shape=(
            jax.ShapeDtypeStruct((nt, TOPK), jnp.int32),
            jax.ShapeDtypeStruct((nt, TOPK), jnp.float32),
            jax.ShapeDtypeStruct((CH, NMETA), jnp.int32),
        ),
        scratch_shapes=[pltpu.VMEM((nt, NE), jnp.float32),
                        pltpu.VMEM((nt, NE), jnp.float32)],
    )(xf, gate_w.astype(jnp.float32))

    flat_pos = pos.reshape(-1)
    sorted_a = jnp.zeros((nr,), jnp.int32).at[flat_pos].set(
        jnp.arange(nr, dtype=jnp.int32))
    a_sorted = jnp.take(xf, sorted_a >> 1, axis=0)

    gws = shared_gate_w.reshape(2, INTER, DIM)
    ups = shared_up_w.reshape(2, INTER, DIM)
    dws = shared_down_w.reshape(DIM, 2, INTER).transpose(1, 0, 2)

    grid_spec = pltpu.PrefetchScalarGridSpec(
        num_scalar_prefetch=1,
        grid=(steps,),
        in_specs=[
            pl.BlockSpec((M, DIM), lambda t, mt: (mt[t, 0], 0)),
            pl.BlockSpec((M, DIM), lambda t, mt: (mt[t, 1], 0)),
            pl.BlockSpec((1, INTER, DIM), lambda t, mt: (mt[t, 3], 0, 0)),
            pl.BlockSpec((1, INTER, DIM), lambda t, mt: (mt[t, 3], 0, 0)),
            pl.BlockSpec((1, DIM, INTER), lambda t, mt: (mt[t, 3], 0, 0)),
            pl.BlockSpec((1, INTER, DIM), lambda t, mt: (mt[t, 6], 0, 0)),
            pl.BlockSpec((1, INTER, DIM), lambda t, mt: (mt[t, 6], 0, 0)),
            pl.BlockSpec((1, DIM, INTER), lambda t, mt: (mt[t, 6], 0, 0)),
        ],
        out_specs=pl.BlockSpec((M, DIM), lambda t, mt: (mt[t, 2], 0)),
    )
    out = pl.pallas_call(
        _gmm_body,
        grid_spec=grid_spec,
        out_shape=jax.ShapeDtypeStruct((rtot, DIM), jnp.float32),
        compiler_params=pltpu.CompilerParams(
            dimension_semantics=("arbitrary",)),
    )(meta, a_sorted, xf,
      expert_gate_w, expert_up_w, expert_down_w, gws, ups, dws)

    y = (wts[:, 0:1] * jnp.take(out, pos[:, 0], axis=0)
         + wts[:, 1:2] * jnp.take(out, pos[:, 1], axis=0)
         + out[nr:nr + nt] + out[nr + nt:])
    return y.astype(x.dtype).reshape(shape)


# speedup vs baseline: 1.9079x; 1.0493x over previous
import jax
import jax.numpy as jnp
from jax.experimental import pallas as pl
from jax.experimental.pallas import tpu as pltpu

DIM = 1024
INTER = 512
NE = 16
TOPK = 2
NG = 18
M = 256
MSH = M.bit_length() - 1
CH = 128
NMETA = 8


def _router_body(x_ref, gw_ref, pos_ref, wts_ref, meta_ref, cum_ref, oh_ref):
    nt = x_ref.shape[0]
    s = jax.lax.dot_general(x_ref[...], gw_ref[...], (((1,), (1,)), ((), ())),
                            preferred_element_type=jnp.float32)
    m = jnp.max(s, axis=1, keepdims=True)
    p = jnp.exp(s - m)
    sm = p / jnp.sum(p, axis=1, keepdims=True)
    lane = jax.lax.broadcasted_iota(jnp.int32, (nt, NE), 1)
    m1 = jnp.max(sm, axis=1, keepdims=True)
    i1 = jnp.min(jnp.where(sm == m1, lane, NE), axis=1, keepdims=True)
    sm2 = jnp.where(lane == i1, -1.0, sm)
    m2 = jnp.max(sm2, axis=1, keepdims=True)
    i2 = jnp.min(jnp.where(sm2 == m2, lane, NE), axis=1, keepdims=True)
    oh_ref[...] = ((lane == i1) | (lane == i2)).astype(jnp.float32)

    r = jax.lax.broadcasted_iota(jnp.int32, (CH, CH), 0)
    c = jax.lax.broadcasted_iota(jnp.int32, (CH, CH), 1)
    tri = (r >= c).astype(jnp.float32)

    def step(i, carry):
        ch = oh_ref[pl.ds(i * CH, CH), :]
        incl = jax.lax.dot_general(tri, ch, (((1,), (0,)), ((), ())),
                                   preferred_element_type=jnp.float32)
        cum_ref[pl.ds(i * CH, CH), :] = incl - ch + carry
        return carry + incl[CH - 1:CH, :]

    counts = jax.lax.fori_loop(0, nt // CH, step, jnp.zeros((1, NE), jnp.float32))

    lane1 = lane[0:1, :]
    off = jnp.zeros((1, NE), jnp.float32)
    for k in range(NE):
        ck = jnp.sum(jnp.where(lane1 == k, counts, 0.0), axis=1, keepdims=True)
        off = off + jnp.where(lane1 > k, ck, 0.0)

    cum = cum_ref[...]
    offb = jnp.broadcast_to(off, (nt, NE))
    pos1 = jnp.sum(jnp.where(lane == i1, cum + offb, 0.0), axis=1, keepdims=True)
    pos2 = jnp.sum(jnp.where(lane == i2, cum + offb, 0.0), axis=1, keepdims=True)
    pos_ref[...] = jnp.concatenate([pos1, pos2], axis=1).astype(jnp.int32)
    wts_ref[...] = jnp.concatenate([m1, m2], axis=1)

    nr = nt * TOPK
    nb_r = nr // M
    nb_x = nt // M
    nb = (nr + nt) // M
    steps = nb_r + (NE - 1) + nb_x
    ngr = NE + 1
    gl = jax.lax.broadcasted_iota(jnp.int32, (1, 32), 1)
    cnt_i = jnp.zeros((1, 32), jnp.int32)
    for k in range(NE):
        ck = jnp.sum(jnp.where(lane1 == k, counts, 0.0), axis=1, keepdims=True)
        cnt_i = cnt_i + jnp.where(gl == k, ck.astype(jnp.int32), 0)
    sizes = jnp.where(gl < NE, cnt_i,
                      jnp.where(gl < ngr, nt, 0))
    offg = jnp.zeros((1, 32), jnp.int32)
    for k in range(ngr):
        ck = jnp.sum(jnp.where(gl == k, sizes, 0), axis=1, keepdims=True)
        offg = offg + jnp.where(gl > k, ck, 0)
    endg = offg + sizes
    fblk = offg >> MSH
    lblk = (endg - 1) >> MSH
    tiles = jnp.where((sizes > 0) & (gl < ngr), lblk - fblk + 1, 0)
    ctiles = jnp.zeros((1, 32), jnp.int32)
    for k in range(ngr):
        ck = jnp.sum(jnp.where(gl == k, tiles, 0), axis=1, keepdims=True)
        ctiles = ctiles + jnp.where(gl >= k, ck, 0)
    sstart = ctiles - tiles

    tcol = jax.lax.broadcasted_iota(jnp.int32, (CH, 1), 0)
    big = jnp.broadcast_to(jnp.where(gl < ngr, ctiles, 10 ** 9), (CH, 32))
    e_arr = jnp.sum((big <= tcol).astype(jnp.int32), axis=1, keepdims=True)

    def glut(v):
        return jnp.sum(jnp.where(
            jax.lax.broadcasted_iota(jnp.int32, (CH, 32), 1) == e_arr,
            jnp.broadcast_to(v, (CH, 32)), 0), axis=1, keepdims=True)

    valid = (e_arr < ngr) & (tcol < steps)
    j = tcol - glut(sstart)
    m_glob = jnp.where(valid, glut(fblk) + j, nb - 1)
    ms = jnp.minimum(m_glob, nb_r - 1)
    mx = jnp.where(valid & (e_arr == NE), m_glob - nb_r,
                   jnp.where(e_arr > NE, nb_x - 1, 0))
    ew = jnp.minimum(e_arr, NE - 1)
    lo = jnp.where(valid, glut(offg), 0)
    hi = jnp.where(valid, glut(endg), 0)
    sh = (e_arr >= NE).astype(jnp.int32)
    fi = (valid & ((j > 0) | ((glut(offg) & (M - 1)) == 0))).astype(jnp.int32)

    mcol = jax.lax.broadcasted_iota(jnp.int32, (CH, NMETA), 1)
    meta = jnp.where(mcol == 0, ms, 0)
    for k, v in enumerate([mx, m_glob, ew, lo, hi, jnp.zeros_like(fi), fi]):
        meta = meta + jnp.where(mcol == k + 1, v, 0)
    meta_ref[...] = jnp.where(mcol == 7, jnp.where(sh == 1, fi + 2, fi), meta)


def _gmm_body(meta_ref, a_ref, x_ref, gw_ref, up_ref, dw_ref,
              gws_ref, ups_ref, dws_ref, out_ref):
    t = pl.program_id(0)
    rows = meta_ref[t, 2] * M + jax.lax.broadcasted_iota(jnp.int32, (M, 1), 0)
    mask = ((rows >= meta_ref[t, 4]) & (rows < meta_ref[t, 5])
            ).astype(jnp.float32)
    code = meta_ref[t, 7]

    def ffn(a, gw, up, dw):
        hg = jax.lax.dot_general(a, gw, (((1,), (1,)), ((), ())),
                                 preferred_element_type=jnp.float32)
        hu = jax.lax.dot_general(a, up, (((1,), (1,)), ((), ())),
                                 preferred_element_type=jnp.float32)
        h = hg * jax.lax.logistic(hg) * hu * mask
        return jax.lax.dot_general(h, dw, (((1,), (1,)), ((), ())),
                                   preferred_element_type=jnp.float32)

    @pl.when(code == 0)
    def _():
        out_ref[...] += ffn(a_ref[...], gw_ref[0], up_ref[0], dw_ref[0])

    @pl.when(code == 1)
    def _():
        out_ref[...] = ffn(a_ref[...], gw_ref[0], up_ref[0], dw_ref[0])

    def shared_ffn():
        a = x_ref[...]
        return (ffn(a, gws_ref[0], ups_ref[0], dws_ref[0])
                + ffn(a, gws_ref[1], ups_ref[1], dws_ref[1]))

    @pl.when(code == 2)
    def _():
        out_ref[...] += shared_ffn()

    @pl.when(code == 3)
    def _():
        out_ref[...] = shared_ffn()


def kernel(x, gate_w, expert_gate_w, expert_up_w, expert_down_w,
           shared_gate_w, shared_up_w, shared_down_w):
    shape = x.shape
    xf = x.reshape(-1, DIM).astype(jnp.float32)
    nt = xf.shape[0]
    nr = nt * TOPK
    rtot = nr + nt
    nb_r = nr // M
    nb_x = nt // M
    steps = nb_r + (NE - 1) + nb_x

    pos, wts, meta = pl.pallas_call(
        _router_body,
        out_shape=(
            jax.ShapeDtypeStruct((nt, TOPK), jnp.int32),
            jax.ShapeDtypeStruct((nt, TOPK), jnp.float32),
            jax.ShapeDtypeStruct((CH, NMETA), jnp.int32),
        ),
        scratch_shapes=[pltpu.VMEM((nt, NE), jnp.float32),
                        pltpu.VMEM((nt, NE), jnp.float32)],
    )(xf, gate_w.astype(jnp.float32))

    flat_pos = pos.reshape(-1)
    sorted_a = jnp.zeros((nr,), jnp.int32).at[flat_pos].set(
        jnp.arange(nr, dtype=jnp.int32))
    a_sorted = jnp.take(xf, sorted_a >> 1, axis=0)

    gws = shared_gate_w.reshape(2, INTER, DIM)
    ups = shared_up_w.reshape(2, INTER, DIM)
    dws = shared_down_w.reshape(DIM, 2, INTER).transpose(1, 0, 2)

    grid_spec = pltpu.PrefetchScalarGridSpec(
        num_scalar_prefetch=1,
        grid=(steps,),
        in_specs=[
            pl.BlockSpec((M, DIM), lambda t, mt: (mt[t, 0], 0)),
            pl.BlockSpec((M, DIM), lambda t, mt: (mt[t, 1], 0)),
            pl.BlockSpec((1, INTER, DIM), lambda t, mt: (mt[t, 3], 0, 0)),
            pl.BlockSpec((1, INTER, DIM), lambda t, mt: (mt[t, 3], 0, 0)),
            pl.BlockSpec((1, DIM, INTER), lambda t, mt: (mt[t, 3], 0, 0)),
            pl.BlockSpec((2, INTER, DIM), lambda t, mt: (0, 0, 0)),
            pl.BlockSpec((2, INTER, DIM), lambda t, mt: (0, 0, 0)),
            pl.BlockSpec((2, DIM, INTER), lambda t, mt: (0, 0, 0)),
        ],
        out_specs=pl.BlockSpec((M, DIM), lambda t, mt: (mt[t, 2], 0)),
    )
    out = pl.pallas_call(
        _gmm_body,
        grid_spec=grid_spec,
        out_shape=jax.ShapeDtypeStruct((rtot, DIM), jnp.float32),
        compiler_params=pltpu.CompilerParams(
            dimension_semantics=("arbitrary",)),
    )(meta, a_sorted, xf,
      expert_gate_w, expert_up_w, expert_down_w, gws, ups, dws)

    y = (wts[:, 0:1] * jnp.take(out, pos[:, 0], axis=0)
         + wts[:, 1:2] * jnp.take(out, pos[:, 1], axis=0)
         + out[nr:])
    return y.astype(x.dtype).reshape(shape)


# speedup vs baseline: 2.5680x; 1.3460x over previous
import functools

import jax
import jax.numpy as jnp
from jax.experimental import pallas as pl
from jax.experimental.pallas import tpu as pltpu
from jax.experimental.pallas import tpu_sc as plsc

DIM = 1024
INTER = 512
NE = 16
TOPK = 2
NG = 18
M = 256
MSH = M.bit_length() - 1
CH = 128
NMETA = 8


def _router_body(x_ref, gw_ref, pos_ref, wts_ref, meta_ref, cum_ref, oh_ref):
    nt = x_ref.shape[0]
    s = jax.lax.dot_general(x_ref[...], gw_ref[...], (((1,), (1,)), ((), ())),
                            preferred_element_type=jnp.float32)
    m = jnp.max(s, axis=1, keepdims=True)
    p = jnp.exp(s - m)
    sm = p / jnp.sum(p, axis=1, keepdims=True)
    lane = jax.lax.broadcasted_iota(jnp.int32, (nt, NE), 1)
    m1 = jnp.max(sm, axis=1, keepdims=True)
    i1 = jnp.min(jnp.where(sm == m1, lane, NE), axis=1, keepdims=True)
    sm2 = jnp.where(lane == i1, -1.0, sm)
    m2 = jnp.max(sm2, axis=1, keepdims=True)
    i2 = jnp.min(jnp.where(sm2 == m2, lane, NE), axis=1, keepdims=True)
    oh_ref[...] = ((lane == i1) | (lane == i2)).astype(jnp.float32)

    r = jax.lax.broadcasted_iota(jnp.int32, (CH, CH), 0)
    c = jax.lax.broadcasted_iota(jnp.int32, (CH, CH), 1)
    tri = (r >= c).astype(jnp.float32)

    def step(i, carry):
        ch = oh_ref[pl.ds(i * CH, CH), :]
        incl = jax.lax.dot_general(tri, ch, (((1,), (0,)), ((), ())),
                                   preferred_element_type=jnp.float32)
        cum_ref[pl.ds(i * CH, CH), :] = incl - ch + carry
        return carry + incl[CH - 1:CH, :]

    counts = jax.lax.fori_loop(0, nt // CH, step, jnp.zeros((1, NE), jnp.float32))

    lane1 = lane[0:1, :]
    off = jnp.zeros((1, NE), jnp.float32)
    for k in range(NE):
        ck = jnp.sum(jnp.where(lane1 == k, counts, 0.0), axis=1, keepdims=True)
        off = off + jnp.where(lane1 > k, ck, 0.0)

    cum = cum_ref[...]
    offb = jnp.broadcast_to(off, (nt, NE))
    pos1 = jnp.sum(jnp.where(lane == i1, cum + offb, 0.0), axis=1, keepdims=True)
    pos2 = jnp.sum(jnp.where(lane == i2, cum + offb, 0.0), axis=1, keepdims=True)
    pos_ref[...] = jnp.concatenate([pos1, pos2], axis=1).astype(jnp.int32)
    wts_ref[...] = jnp.concatenate([m1, m2], axis=1)

    nr = nt * TOPK
    nb_r = nr // M
    nb_x = nt // M
    nb = (nr + nt) // M
    steps = nb_r + (NE - 1) + nb_x
    ngr = NE + 1
    gl = jax.lax.broadcasted_iota(jnp.int32, (1, 32), 1)
    cnt_i = jnp.zeros((1, 32), jnp.int32)
    for k in range(NE):
        ck = jnp.sum(jnp.where(lane1 == k, counts, 0.0), axis=1, keepdims=True)
        cnt_i = cnt_i + jnp.where(gl == k, ck.astype(jnp.int32), 0)
    sizes = jnp.where(gl < NE, cnt_i,
                      jnp.where(gl < ngr, nt, 0))
    offg = jnp.zeros((1, 32), jnp.int32)
    for k in range(ngr):
        ck = jnp.sum(jnp.where(gl == k, sizes, 0), axis=1, keepdims=True)
        offg = offg + jnp.where(gl > k, ck, 0)
    endg = offg + sizes
    fblk = offg >> MSH
    lblk = (endg - 1) >> MSH
    tiles = jnp.where((sizes > 0) & (gl < ngr), lblk - fblk + 1, 0)
    ctiles = jnp.zeros((1, 32), jnp.int32)
    for k in range(ngr):
        ck = jnp.sum(jnp.where(gl == k, tiles, 0), axis=1, keepdims=True)
        ctiles = ctiles + jnp.where(gl >= k, ck, 0)
    sstart = ctiles - tiles

    tcol = jax.lax.broadcasted_iota(jnp.int32, (CH, 1), 0)
    big = jnp.broadcast_to(jnp.where(gl < ngr, ctiles, 10 ** 9), (CH, 32))
    e_arr = jnp.sum((big <= tcol).astype(jnp.int32), axis=1, keepdims=True)

    def glut(v):
        return jnp.sum(jnp.where(
            jax.lax.broadcasted_iota(jnp.int32, (CH, 32), 1) == e_arr,
            jnp.broadcast_to(v, (CH, 32)), 0), axis=1, keepdims=True)

    valid = (e_arr < ngr) & (tcol < steps)
    j = tcol - glut(sstart)
    m_glob = jnp.where(valid, glut(fblk) + j, nb - 1)
    ms = jnp.minimum(m_glob, nb_r - 1)
    mx = jnp.where(valid & (e_arr == NE), m_glob - nb_r,
                   jnp.where(e_arr > NE, nb_x - 1, 0))
    ew = jnp.minimum(e_arr, NE - 1)
    lo = jnp.where(valid, glut(offg), 0)
    hi = jnp.where(valid, glut(endg), 0)
    sh = (e_arr >= NE).astype(jnp.int32)
    fi = (valid & ((j > 0) | ((glut(offg) & (M - 1)) == 0))).astype(jnp.int32)

    mcol = jax.lax.broadcasted_iota(jnp.int32, (CH, NMETA), 1)
    meta = jnp.where(mcol == 0, ms, 0)
    for k, v in enumerate([mx, m_glob, ew, lo, hi, jnp.zeros_like(fi), fi]):
        meta = meta + jnp.where(mcol == k + 1, v, 0)
    meta_ref[...] = jnp.where(mcol == 7, jnp.where(sh == 1, fi + 2, fi), meta)


_SC_NC = 2
_SC_NS = 16
_NW = _SC_NC * _SC_NS


def _dispatch_body(x_hbm, p0_hbm, p1_hbm, a_hbm, xrows, idx0, idx1, sem):
    w = jax.lax.axis_index("s") * _SC_NC + jax.lax.axis_index("c")
    nt = x_hbm.shape[0]
    tpw = nt // _NW
    tc = 64
    base = w * tpw
    for cidx in range(tpw // tc):
        tb = base + cidx * tc
        pltpu.sync_copy(x_hbm.at[pl.ds(tb, tc)], xrows)
        pltpu.sync_copy(p0_hbm.at[pl.ds(tb, tc)], idx0)
        pltpu.sync_copy(p1_hbm.at[pl.ds(tb, tc)], idx1)
        c0 = pltpu.async_copy(xrows, a_hbm.at[idx0], sem)
        c1 = pltpu.async_copy(xrows, a_hbm.at[idx1], sem)
        c0.wait()
        c1.wait()


def _gmm_body(meta_ref, a_ref, x_ref, gw_ref, up_ref, dw_ref,
              gws_ref, ups_ref, dws_ref, out_ref):
    t = pl.program_id(0)
    rows = meta_ref[t, 2] * M + jax.lax.broadcasted_iota(jnp.int32, (M, 1), 0)
    mask = ((rows >= meta_ref[t, 4]) & (rows < meta_ref[t, 5])
            ).astype(jnp.float32)
    code = meta_ref[t, 7]

    def ffn(a, gw, up, dw):
        hg = jax.lax.dot_general(a, gw, (((1,), (1,)), ((), ())),
                                 preferred_element_type=jnp.float32)
        hu = jax.lax.dot_general(a, up, (((1,), (1,)), ((), ())),
                                 preferred_element_type=jnp.float32)
        h = hg * jax.lax.logistic(hg) * hu * mask
        return jax.lax.dot_general(h, dw, (((1,), (1,)), ((), ())),
                                   preferred_element_type=jnp.float32)

    @pl.when(code == 0)
    def _():
        out_ref[...] += ffn(a_ref[...], gw_ref[0], up_ref[0], dw_ref[0])

    @pl.when(code == 1)
    def _():
        out_ref[...] = ffn(a_ref[...], gw_ref[0], up_ref[0], dw_ref[0])

    def shared_ffn():
        a = x_ref[...]
        return (ffn(a, gws_ref[0], ups_ref[0], dws_ref[0])
                + ffn(a, gws_ref[1], ups_ref[1], dws_ref[1]))

    @pl.when(code == 2)
    def _():
        out_ref[...] += shared_ffn()

    @pl.when(code == 3)
    def _():
        out_ref[...] = shared_ffn()


def kernel(x, gate_w, expert_gate_w, expert_up_w, expert_down_w,
           shared_gate_w, shared_up_w, shared_down_w):
    shape = x.shape
    xf = x.reshape(-1, DIM).astype(jnp.float32)
    nt = xf.shape[0]
    nr = nt * TOPK
    rtot = nr + nt
    nb_r = nr // M
    nb_x = nt // M
    steps = nb_r + (NE - 1) + nb_x

    pos, wts, meta = pl.pallas_call(
        _router_body,
        out_shape=(
            jax.ShapeDtypeStruct((nt, TOPK), jnp.int32),
            jax.ShapeDtypeStruct((nt, TOPK), jnp.float32),
            jax.ShapeDtypeStruct((CH, NMETA), jnp.int32),
        ),
        scratch_shapes=[pltpu.VMEM((nt, NE), jnp.float32),
                        pltpu.VMEM((nt, NE), jnp.float32)],
    )(xf, gate_w.astype(jnp.float32))

    pos0 = pos[:, 0]
    pos1 = pos[:, 1]
    dispatch = pl.kernel(
        _dispatch_body,
        out_type=jax.ShapeDtypeStruct((nr, DIM), jnp.float32),
        mesh=plsc.VectorSubcoreMesh(core_axis_name="c", subcore_axis_name="s"),
        scratch_types=[
            pltpu.VMEM((64, DIM), jnp.float32),
            pltpu.VMEM((64,), jnp.int32),
            pltpu.VMEM((64,), jnp.int32),
            pltpu.SemaphoreType.DMA,
        ],
    )
    a_sorted = dispatch(xf, pos0, pos1)

    gws = shared_gate_w.reshape(2, INTER, DIM)
    ups = shared_up_w.reshape(2, INTER, DIM)
    dws = shared_down_w.reshape(DIM, 2, INTER).transpose(1, 0, 2)

    grid_spec = pltpu.PrefetchScalarGridSpec(
        num_scalar_prefetch=1,
        grid=(steps,),
        in_specs=[
            pl.BlockSpec((M, DIM), lambda t, mt: (mt[t, 0], 0)),
            pl.BlockSpec((M, DIM), lambda t, mt: (mt[t, 1], 0)),
            pl.BlockSpec((1, INTER, DIM), lambda t, mt: (mt[t, 3], 0, 0)),
            pl.BlockSpec((1, INTER, DIM), lambda t, mt: (mt[t, 3], 0, 0)),
            pl.BlockSpec((1, DIM, INTER), lambda t, mt: (mt[t, 3], 0, 0)),
            pl.BlockSpec((2, INTER, DIM), lambda t, mt: (0, 0, 0)),
            pl.BlockSpec((2, INTER, DIM), lambda t, mt: (0, 0, 0)),
            pl.BlockSpec((2, DIM, INTER), lambda t, mt: (0, 0, 0)),
        ],
        out_specs=pl.BlockSpec((M, DIM), lambda t, mt: (mt[t, 2], 0)),
    )
    out = pl.pallas_call(
        _gmm_body,
        grid_spec=grid_spec,
        out_shape=jax.ShapeDtypeStruct((rtot, DIM), jnp.float32),
        compiler_params=pltpu.CompilerParams(
            dimension_semantics=("arbitrary",)),
    )(meta, a_sorted, xf,
      expert_gate_w, expert_up_w, expert_down_w, gws, ups, dws)

    y = (wts[:, 0:1] * jnp.take(out, pos[:, 0], axis=0)
         + wts[:, 1:2] * jnp.take(out, pos[:, 1], axis=0)
         + out[nr:])
    return y.astype(x.dtype).reshape(shape)
